# baseline (speedup 1.0000x reference)
import functools

import jax
import jax.numpy as jnp
from jax import lax
from jax.experimental import pallas as pl
from jax.experimental.pallas import tpu as pltpu
from jax.experimental.pallas import tpu_sc as plsc

BATCH = 16384
HIST = 20
EMBED_DIM = 32
NC, NS = 2, 16
NW = NC * NS
B_PER_W = BATCH // NW
LANES = 16


def _transpose_block(buf, tbuf):
    iota = lax.iota(jnp.int32, LANES)

    def col_step(c, _):
        def row_step(j, _):
            r0 = j * LANES
            vec = plsc.load_gather(buf, [iota + r0, jnp.full((LANES,), c)])
            tbuf[c, pl.ds(r0, LANES)] = vec
            return _

        lax.fori_loop(0, B_PER_W // LANES, row_step, 0, unroll=4)
        return _

    lax.fori_loop(0, EMBED_DIM, col_step, 0)


def _gather_body(idx_hbm, table_hbm, out_hbm, idx_v, rows0, rows1, t0, t1,
                 sem_i, sem_g, sem_o):
    wid = lax.axis_index("s") * NC + lax.axis_index("c")
    b0 = wid * B_PER_W

    pltpu.async_copy(idx_hbm.at[:, pl.ds(b0, B_PER_W)], idx_v, sem_i).wait()

    bufs = (rows0, rows1)
    tbufs = (t0, t1)
    gathers = [None] * HIST
    writes = [None] * HIST
    for h in range(HIST):
        gathers[h] = pltpu.async_copy(
            table_hbm.at[idx_v.at[h]], bufs[h % 2], sem_g)
        if h >= 1:
            gathers[h - 1].wait()
            if h >= 2:
                writes[h - 2].wait()
            _transpose_block(bufs[(h - 1) % 2], tbufs[(h - 1) % 2])
            writes[h - 1] = pltpu.async_copy(
                tbufs[(h - 1) % 2],
                out_hbm.at[h - 1, :, pl.ds(b0, B_PER_W)], sem_o)
    gathers[HIST - 1].wait()
    writes[HIST - 2].wait()
    _transpose_block(bufs[(HIST - 1) % 2], tbufs[(HIST - 1) % 2])
    writes[HIST - 1] = pltpu.async_copy(
        tbufs[(HIST - 1) % 2],
        out_hbm.at[HIST - 1, :, pl.ds(b0, B_PER_W)], sem_o)
    writes[HIST - 1].wait()


@functools.partial(jax.jit, static_argnames=())
def kernel(indices, table):
    idx_t = indices.astype(jnp.int32).T
    mesh = plsc.VectorSubcoreMesh(
        core_axis_name="c", subcore_axis_name="s",
        num_cores=NC, num_subcores=NS,
    )
    run = pl.kernel(
        _gather_body,
        out_type=jax.ShapeDtypeStruct((HIST, EMBED_DIM, BATCH), jnp.float32),
        mesh=mesh,
        scratch_types=[
            pltpu.VMEM((HIST, B_PER_W), jnp.int32),
            pltpu.VMEM((B_PER_W, EMBED_DIM), jnp.float32),
            pltpu.VMEM((B_PER_W, EMBED_DIM), jnp.float32),
            pltpu.VMEM((EMBED_DIM, B_PER_W), jnp.float32),
            pltpu.VMEM((EMBED_DIM, B_PER_W), jnp.float32),
            pltpu.SemaphoreType.DMA,
            pltpu.SemaphoreType.DMA,
            pltpu.SemaphoreType.DMA,
        ],
        compiler_params=pltpu.CompilerParams(
            use_tc_tiling_on_sc=False, needs_layout_passes=False),
    )
    out_t = run(idx_t, table)
    return out_t.transpose(2, 0, 1)


# speedup vs baseline: 1.1296x; 1.1296x over previous
import functools

import jax
import jax.numpy as jnp
from jax import lax
from jax.experimental import pallas as pl
from jax.experimental.pallas import tpu as pltpu
from jax.experimental.pallas import tpu_sc as plsc

BATCH = 16384
HIST = 20
EMBED_DIM = 32
NC, NS = 2, 16
NW = NC * NS
B_PER_W = BATCH // NW


def _gather_body(*refs):
    idx_hbms = refs[:HIST]
    table_hbm = refs[HIST]
    out_hbm = refs[HIST + 1]
    idx_v, rows0, rows1, sem_i, sem_g, sem_o = refs[HIST + 2:]

    wid = lax.axis_index("s") * NC + lax.axis_index("c")
    b0 = wid * B_PER_W

    for h in range(HIST):
        pltpu.async_copy(idx_hbms[h].at[pl.ds(b0, B_PER_W)], idx_v.at[h],
                         sem_i)
    for h in range(HIST):
        pltpu.make_async_copy(idx_hbms[h].at[pl.ds(b0, B_PER_W)],
                              idx_v.at[h], sem_i).wait()

    bufs = (rows0, rows1)
    gathers = [None] * HIST
    writes = [None] * HIST
    for h in range(HIST):
        gathers[h] = pltpu.async_copy(
            table_hbm.at[idx_v.at[h]], bufs[h % 2], sem_g)
        if h >= 1:
            if h >= 2:
                writes[h - 2].wait()
            gathers[h - 1].wait()
            writes[h - 1] = pltpu.async_copy(
                bufs[(h - 1) % 2],
                out_hbm.at[pl.ds(b0, B_PER_W), h - 1], sem_o)
    gathers[HIST - 1].wait()
    writes[HIST - 2].wait()
    writes[HIST - 1] = pltpu.async_copy(
        bufs[(HIST - 1) % 2],
        out_hbm.at[pl.ds(b0, B_PER_W), HIST - 1], sem_o)
    writes[HIST - 1].wait()


@functools.partial(jax.jit, static_argnames=())
def kernel(indices, table):
    idx = indices.astype(jnp.int32)
    cols = tuple(idx[:, h] for h in range(HIST))
    mesh = plsc.VectorSubcoreMesh(
        core_axis_name="c", subcore_axis_name="s",
        num_cores=NC, num_subcores=NS,
    )
    run = pl.kernel(
        _gather_body,
        out_type=jax.ShapeDtypeStruct((BATCH, HIST, EMBED_DIM), jnp.float32),
        mesh=mesh,
        scratch_types=[
            pltpu.VMEM((HIST, B_PER_W), jnp.int32),
            pltpu.VMEM((B_PER_W, EMBED_DIM), jnp.float32),
            pltpu.VMEM((B_PER_W, EMBED_DIM), jnp.float32),
            pltpu.SemaphoreType.DMA,
            pltpu.SemaphoreType.DMA,
            pltpu.SemaphoreType.DMA,
        ],
        compiler_params=pltpu.CompilerParams(
            use_tc_tiling_on_sc=False, needs_layout_passes=False),
    )
    return run(*cols, table)


# speedup vs baseline: 1.1303x; 1.0006x over previous
import functools

import jax
import jax.numpy as jnp
from jax import lax
from jax.experimental import pallas as pl
from jax.experimental.pallas import tpu as pltpu
from jax.experimental.pallas import tpu_sc as plsc

BATCH = 16384
HIST = 20
EMBED_DIM = 32
NUM_ROWS = BATCH * HIST
NC, NS = 2, 16
NW = NC * NS
B_PER_W = BATCH // NW
ROWS_PER_W = NUM_ROWS // NW
CHUNK = 1024
N_CHUNKS = ROWS_PER_W // CHUNK
LANES = 16
HIST_PAD = 24


def _stage_body(idxt_hbm, out_hbm, ibuf, obuf, sem):
    wid = lax.axis_index("s") * NC + lax.axis_index("c")
    b0 = wid * B_PER_W

    pltpu.async_copy(idxt_hbm.at[pl.ds(0, 16), pl.ds(b0, B_PER_W)],
                     ibuf.at[pl.ds(0, 16)], sem)
    pltpu.async_copy(idxt_hbm.at[pl.ds(16, 4), pl.ds(b0, B_PER_W)],
                     ibuf.at[pl.ds(16, 4)], sem)
    pltpu.make_async_copy(idxt_hbm.at[pl.ds(0, 16), pl.ds(b0, B_PER_W)],
                          ibuf.at[pl.ds(0, 16)], sem).wait()
    pltpu.make_async_copy(idxt_hbm.at[pl.ds(16, 4), pl.ds(b0, B_PER_W)],
                          ibuf.at[pl.ds(16, 4)], sem).wait()

    iota_h = lax.iota(jnp.int32, LANES) * HIST
    for h in range(HIST):
        def row_step(j, _, h=h):
            r0 = j * LANES
            vec = ibuf[h, pl.ds(r0, LANES)]
            plsc.store_scatter(obuf, [iota_h + (r0 * HIST + h)], vec)
            return _

        lax.fori_loop(0, B_PER_W // LANES, row_step, 0, unroll=4)

    pltpu.async_copy(obuf, out_hbm.at[pl.ds(wid * ROWS_PER_W, ROWS_PER_W)], sem).wait()


def _gather_body(idx_hbm, table_hbm, out_hbm, idx_v, rows0, rows1, sem_i,
                 sem_g, sem_o):
    wid = lax.axis_index("s") * NC + lax.axis_index("c")
    base = wid * ROWS_PER_W

    pltpu.async_copy(idx_hbm.at[wid], idx_v, sem_i).wait()

    bufs = (rows0, rows1)
    gathers = [None] * N_CHUNKS
    writes = [None] * N_CHUNKS
    for g in range(N_CHUNKS):
        gathers[g] = pltpu.async_copy(
            table_hbm.at[idx_v.at[pl.ds(g * CHUNK, CHUNK)]], bufs[g % 2],
            sem_g)
        if g >= 1:
            if g >= 2:
                writes[g - 2].wait()
            gathers[g - 1].wait()
            writes[g - 1] = pltpu.async_copy(
                bufs[(g - 1) % 2],
                out_hbm.at[pl.ds(base + (g - 1) * CHUNK, CHUNK)], sem_o)
    gathers[N_CHUNKS - 1].wait()
    writes[N_CHUNKS - 2].wait()
    writes[N_CHUNKS - 1] = pltpu.async_copy(
        bufs[(N_CHUNKS - 1) % 2],
        out_hbm.at[pl.ds(base + (N_CHUNKS - 1) * CHUNK, CHUNK)], sem_o)
    writes[N_CHUNKS - 1].wait()


@functools.partial(jax.jit, static_argnames=())
def kernel(indices, table):
    idx_t = indices.astype(jnp.int32).T
    mesh = plsc.VectorSubcoreMesh(
        core_axis_name="c", subcore_axis_name="s",
        num_cores=NC, num_subcores=NS,
    )
    stage = pl.kernel(
        _stage_body,
        out_type=jax.ShapeDtypeStruct((NUM_ROWS,), jnp.int32),
        mesh=mesh,
        scratch_types=[
            pltpu.VMEM((HIST_PAD, B_PER_W), jnp.int32),
            pltpu.VMEM((ROWS_PER_W,), jnp.int32),
            pltpu.SemaphoreType.DMA,
        ],
        compiler_params=pltpu.CompilerParams(
            use_tc_tiling_on_sc=True, needs_layout_passes=False),
    )
    idx_flat = stage(idx_t).reshape(NW, ROWS_PER_W)
    run = pl.kernel(
        _gather_body,
        out_type=jax.ShapeDtypeStruct((NUM_ROWS, EMBED_DIM), jnp.float32),
        mesh=mesh,
        scratch_types=[
            pltpu.VMEM((ROWS_PER_W,), jnp.int32),
            pltpu.VMEM((CHUNK, EMBED_DIM), jnp.float32),
            pltpu.VMEM((CHUNK, EMBED_DIM), jnp.float32),
            pltpu.SemaphoreType.DMA,
            pltpu.SemaphoreType.DMA,
            pltpu.SemaphoreType.DMA,
        ],
        compiler_params=pltpu.CompilerParams(
            use_tc_tiling_on_sc=False, needs_layout_passes=False),
    )
    out = run(idx_flat, table)
    return out.reshape(BATCH, HIST, EMBED_DIM)
